# baseline (speedup 1.0000x reference)
import functools

import jax
import jax.numpy as jnp
from jax import lax
from jax.experimental import pallas as pl
from jax.experimental.pallas import tpu as pltpu
from jax.experimental.pallas import tpu_sc as plsc

VOCAB = 1000000
D = 64
B = 16384
H = 128
QUART = 1 << 18
WIDE = 2 * D

NC = 2
NS = 16
NW = NC * NS
B_PER_W = B // NW
CHUNK = 128
N_CHUNKS = B_PER_W // CHUNK

TBLK = 16384
N_TBLK = QUART // TBLK
LAST_BLK = (VOCAB - 1) // TBLK


def _round_bits_u32(x):
    u = lax.bitcast_convert_type(x, jnp.uint32)
    return u + jnp.uint32(0x8000)


def _tp_body(q0_ref, q1_ref, q2_ref, q3_ref, out_ref):
    lo01 = _round_bits_u32(q0_ref[...]) >> jnp.uint32(16)
    hi01 = _round_bits_u32(q1_ref[...]) & jnp.uint32(0xFFFF0000)
    lo23 = _round_bits_u32(q2_ref[...]) >> jnp.uint32(16)
    hi23 = _round_bits_u32(q3_ref[...]) & jnp.uint32(0xFFFF0000)
    p01 = lax.bitcast_convert_type(lo01 | hi01, jnp.float32)
    p23 = lax.bitcast_convert_type(lo23 | hi23, jnp.float32)
    x = jnp.concatenate([p01, p23], axis=0)
    out_ref[...] = x.T


def _tc_transpose_pack(tableT):
    return pl.pallas_call(
        _tp_body,
        grid=(N_TBLK,),
        in_specs=[
            pl.BlockSpec((D, TBLK), lambda i: (0, i)),
            pl.BlockSpec((D, TBLK), lambda i: (0, i + N_TBLK)),
            pl.BlockSpec((D, TBLK), lambda i: (0, i + 2 * N_TBLK)),
            pl.BlockSpec(
                (D, TBLK),
                lambda i: (0, jnp.minimum(i + 3 * N_TBLK, LAST_BLK)),
            ),
        ],
        out_specs=pl.BlockSpec((TBLK, WIDE), lambda i: (i, 0)),
        out_shape=jax.ShapeDtypeStruct((QUART, WIDE), jnp.float32),
        compiler_params=pltpu.CompilerParams(
            dimension_semantics=("arbitrary",),
        ),
    )(tableT, tableT, tableT, tableT)


def _sc_gather_wide(table_wide, uid):
    mesh = plsc.VectorSubcoreMesh(core_axis_name="c", subcore_axis_name="s")

    @functools.partial(
        pl.kernel,
        mesh=mesh,
        out_type=jax.ShapeDtypeStruct((B, WIDE), jnp.float32),
        scratch_types=[
            pltpu.VMEM((N_CHUNKS, CHUNK), jnp.int32),
            pltpu.VMEM((B_PER_W, WIDE), jnp.float32),
            pltpu.SemaphoreType.DMA,
        ],
    )
    def k(table_hbm, idx_hbm, out_hbm, idx_v, rows_v, sem):
        wid = lax.axis_index("s") * NC + lax.axis_index("c")
        base = wid * B_PER_W
        idx_copies = [
            pltpu.async_copy(
                idx_hbm.at[pl.ds(base + j * CHUNK, CHUNK)], idx_v.at[j], sem
            )
            for j in range(N_CHUNKS)
        ]
        for c in idx_copies:
            c.wait()

        @pl.loop(0, N_CHUNKS)
        def _(j):
            @pl.loop(0, CHUNK, step=16)
            def _(c):
                slc = (j, pl.ds(c, 16))
                idx_v.at[*slc][...] = (
                    idx_v.at[*slc][...] & jnp.int32(QUART - 1)
                )

        copies = [
            pltpu.async_copy(
                table_hbm.at[idx_v.at[j]],
                rows_v.at[pl.ds(j * CHUNK, CHUNK)],
                sem,
            )
            for j in range(N_CHUNKS)
        ]
        for c in copies:
            c.wait()
        pltpu.sync_copy(rows_v, out_hbm.at[pl.ds(base, B_PER_W)])

    return k(table_wide, uid)


def _mlp_body(wide_ref, q_ref, w1_ref, b1_ref, w2_ref, b2_ref, outT_ref):
    u = lax.bitcast_convert_type(wide_ref[...], jnp.uint32)
    q = q_ref[...].astype(jnp.int32)
    sel_u = jnp.where(
        (q & 1) != 0, u & jnp.uint32(0xFFFF0000), u << jnp.uint32(16)
    )
    sel = lax.bitcast_convert_type(sel_u, jnp.float32)
    emb = jnp.where((q >> 1) != 0, sel[:, D:], sel[:, :D])
    h = jnp.dot(
        emb.astype(jnp.bfloat16), w1_ref[...],
        preferred_element_type=jnp.float32,
    )
    h = jnp.maximum(h + b1_ref[...], 0.0)
    outT = lax.dot_general(
        w2_ref[...], h.astype(jnp.bfloat16),
        dimension_numbers=(((0,), (1,)), ((), ())),
        preferred_element_type=jnp.float32,
    )
    outT_ref[...] = outT + b2_ref[...]


def _tc_mlp(wide, q, W1, b1, W2, b2):
    BLK = 2048
    outT = pl.pallas_call(
        _mlp_body,
        grid=(B // BLK,),
        in_specs=[
            pl.BlockSpec((BLK, WIDE), lambda i: (i, 0)),
            pl.BlockSpec((BLK, 1), lambda i: (i, 0)),
            pl.BlockSpec((D, H), lambda i: (0, 0)),
            pl.BlockSpec((1, H), lambda i: (0, 0)),
            pl.BlockSpec((H, D), lambda i: (0, 0)),
            pl.BlockSpec((D, 1), lambda i: (0, 0)),
        ],
        out_specs=pl.BlockSpec((D, BLK), lambda i: (0, i)),
        out_shape=jax.ShapeDtypeStruct((D, B), jnp.float32),
    )(
        wide, q, W1.astype(jnp.bfloat16), b1.reshape(1, H),
        W2.astype(jnp.bfloat16), b2.reshape(D, 1),
    )
    return outT.T


def kernel(user_id, table, W1, b1, W2, b2):
    uid = user_id.astype(jnp.int32)
    tableT = table.T
    wide_tbl = _tc_transpose_pack(tableT)
    q = (uid >> 18).astype(jnp.int8).reshape(B, 1)
    wide = _sc_gather_wide(wide_tbl, uid)
    return _tc_mlp(wide, q, W1, b1, W2, b2)


# speedup vs baseline: 1.0193x; 1.0193x over previous
import functools

import jax
import jax.numpy as jnp
from jax import lax
from jax.experimental import pallas as pl
from jax.experimental.pallas import tpu as pltpu
from jax.experimental.pallas import tpu_sc as plsc

VOCAB = 1000000
D = 64
B = 16384
H = 128
QUART = 1 << 18
WIDE = 2 * D

NC = 2
NS = 16
NW = NC * NS
B_PER_W = B // NW
CHUNK = 128
N_CHUNKS = B_PER_W // CHUNK

TBLK = 16384
N_TBLK = QUART // TBLK
LAST_BLK = (VOCAB - 1) // TBLK


def _round_bits_u32(x):
    u = lax.bitcast_convert_type(x, jnp.uint32)
    return u + jnp.uint32(0x8000)


def _tp_body(q0_ref, q1_ref, q2_ref, q3_ref, out_ref):
    lo01 = _round_bits_u32(q0_ref[...]) >> jnp.uint32(16)
    hi01 = _round_bits_u32(q1_ref[...]) & jnp.uint32(0xFFFF0000)
    lo23 = _round_bits_u32(q2_ref[...]) >> jnp.uint32(16)
    hi23 = _round_bits_u32(q3_ref[...]) & jnp.uint32(0xFFFF0000)
    p01 = lax.bitcast_convert_type(lo01 | hi01, jnp.float32)
    p23 = lax.bitcast_convert_type(lo23 | hi23, jnp.float32)
    x = jnp.concatenate([p01, p23], axis=0)
    out_ref[...] = x.T


def _tc_transpose_pack(tableT):
    return pl.pallas_call(
        _tp_body,
        grid=(N_TBLK,),
        in_specs=[
            pl.BlockSpec((D, TBLK), lambda i: (0, i)),
            pl.BlockSpec((D, TBLK), lambda i: (0, i + N_TBLK)),
            pl.BlockSpec((D, TBLK), lambda i: (0, i + 2 * N_TBLK)),
            pl.BlockSpec(
                (D, TBLK),
                lambda i: (0, jnp.minimum(i + 3 * N_TBLK, LAST_BLK)),
            ),
        ],
        out_specs=pl.BlockSpec((TBLK, WIDE), lambda i: (i, 0)),
        out_shape=jax.ShapeDtypeStruct((QUART, WIDE), jnp.float32),
        compiler_params=pltpu.CompilerParams(
            dimension_semantics=("arbitrary",),
        ),
    )(tableT, tableT, tableT, tableT)


def _sc_gather_wide(table_wide, uid):
    mesh = plsc.VectorSubcoreMesh(core_axis_name="c", subcore_axis_name="s")

    @functools.partial(
        pl.kernel,
        mesh=mesh,
        out_type=jax.ShapeDtypeStruct((B, WIDE), jnp.float32),
        scratch_types=[
            pltpu.VMEM((N_CHUNKS, CHUNK), jnp.int32),
            pltpu.VMEM((B_PER_W, WIDE), jnp.float32),
            pltpu.SemaphoreType.DMA,
        ],
    )
    def k(table_hbm, idx_hbm, out_hbm, idx_v, rows_v, sem):
        wid = lax.axis_index("s") * NC + lax.axis_index("c")
        base = wid * B_PER_W
        idx_copies = [
            pltpu.async_copy(
                idx_hbm.at[pl.ds(base + j * CHUNK, CHUNK)], idx_v.at[j], sem
            )
            for j in range(N_CHUNKS)
        ]
        for c in idx_copies:
            c.wait()

        @pl.loop(0, N_CHUNKS)
        def _(j):
            @pl.loop(0, CHUNK, step=16)
            def _(c):
                slc = (j, pl.ds(c, 16))
                idx_v.at[*slc][...] = (
                    idx_v.at[*slc][...] & jnp.int32(QUART - 1)
                )

        copies = [
            pltpu.async_copy(
                table_hbm.at[idx_v.at[j]],
                rows_v.at[pl.ds(j * CHUNK, CHUNK)],
                sem,
            )
            for j in range(N_CHUNKS)
        ]
        for c in copies:
            c.wait()
        pltpu.sync_copy(rows_v, out_hbm.at[pl.ds(base, B_PER_W)])

    return k(table_wide, uid)


def _mlp_body(wide_ref, q_ref, w1_ref, b1_ref, w2_ref, b2_ref, outT_ref):
    u = lax.bitcast_convert_type(wide_ref[...], jnp.uint32)
    q = q_ref[...].astype(jnp.int32)
    sel_u = jnp.where(
        (q & 1) != 0, u & jnp.uint32(0xFFFF0000), u << jnp.uint32(16)
    )
    sel = lax.bitcast_convert_type(sel_u, jnp.float32)
    emb = jnp.where((q >> 1) != 0, sel[:, D:], sel[:, :D])
    h = jnp.dot(
        emb.astype(jnp.bfloat16), w1_ref[...],
        preferred_element_type=jnp.float32,
    )
    h = jnp.maximum(h + b1_ref[...], 0.0)
    outT = lax.dot_general(
        w2_ref[...], h.astype(jnp.bfloat16),
        dimension_numbers=(((0,), (1,)), ((), ())),
        preferred_element_type=jnp.float32,
    )
    outT_ref[...] = outT + b2_ref[...]


def _tc_mlp(wide, q, W1, b1, W2, b2):
    BLK = 4096
    outT = pl.pallas_call(
        _mlp_body,
        grid=(B // BLK,),
        in_specs=[
            pl.BlockSpec((BLK, WIDE), lambda i: (i, 0)),
            pl.BlockSpec((BLK, 1), lambda i: (i, 0)),
            pl.BlockSpec((D, H), lambda i: (0, 0)),
            pl.BlockSpec((1, H), lambda i: (0, 0)),
            pl.BlockSpec((H, D), lambda i: (0, 0)),
            pl.BlockSpec((D, 1), lambda i: (0, 0)),
        ],
        out_specs=pl.BlockSpec((D, BLK), lambda i: (0, i)),
        out_shape=jax.ShapeDtypeStruct((D, B), jnp.float32),
    )(
        wide, q, W1.astype(jnp.bfloat16), b1.reshape(1, H),
        W2.astype(jnp.bfloat16), b2.reshape(D, 1),
    )
    return outT.T


def kernel(user_id, table, W1, b1, W2, b2):
    uid = user_id.astype(jnp.int32)
    tableT = table.T
    wide_tbl = _tc_transpose_pack(tableT)
    q = (uid >> 18).astype(jnp.int8).reshape(B, 1)
    wide = _sc_gather_wide(wide_tbl, uid)
    return _tc_mlp(wide, q, W1, b1, W2, b2)
